# Initial kernel scaffold; baseline (speedup 1.0000x reference)
import jax
import jax.numpy as jnp
from jax.experimental import pallas as pl


def kernel(shape_id, colour_id, pos_id, edge_index, batch, shape_emb, col_emb, pos_emb, W1l, b1l, W1r, g1, bt1, W2l, b2l, W2r, g2, bt2, lin_W, lin_b):
    raise NotImplementedError("write your pallas kernel here")


# speedup vs baseline: 4.4685x; 4.4685x over previous
import functools

import jax
import jax.numpy as jnp
from jax import lax
from jax.experimental import pallas as pl
from jax.experimental.pallas import tpu as pltpu
from jax.experimental.pallas import tpu_sc as plsc

N = 10000
E = 320000
HID = 128
NG = 64
NCLS = 2
EPS = 1e-5

NCORES = 2
NSUB = 16
NW = NCORES * NSUB

RPW = 320
ECH = 80

EPW = E // NW
CH = 80
NCH = EPW // CH
TSPAN = 640
ZR = 80

BN_BLK = 1000
GRID = N // BN_BLK


def _vsc_mesh():
  return plsc.VectorSubcoreMesh(core_axis_name="c", subcore_axis_name="s",
                                num_cores=NCORES, num_subcores=NSUB)


def _sc_embed(shape_id, colour_id, pos_id, shape_emb, col_emb, pos_emb):
  @functools.partial(
      pl.kernel,
      out_type=jax.ShapeDtypeStruct((N, HID), jnp.float32),
      mesh=_vsc_mesh(),
      scratch_types=[
          pltpu.VMEM((ECH,), jnp.int32),
          pltpu.VMEM((ECH,), jnp.int32),
          pltpu.VMEM((ECH,), jnp.int32),
          pltpu.VMEM((ECH, HID), jnp.float32),
          pltpu.VMEM((ECH, HID), jnp.float32),
          pltpu.VMEM((ECH, HID), jnp.float32),
          pltpu.SemaphoreType.DMA,
      ],
  )
  def k(sid_h, cid_h, pid_h, semb_h, cemb_h, pemb_h, x_h,
        sidv, cidv, pidv, ra, rb, rc, sem):
    wid = lax.axis_index("s") * NCORES + lax.axis_index("c")
    base = jnp.minimum(wid * RPW, N - RPW)
    for cc in range(RPW // ECH):
      off = base + cc * ECH
      pltpu.sync_copy(sid_h.at[pl.ds(off, ECH)], sidv)
      pltpu.sync_copy(cid_h.at[pl.ds(off, ECH)], cidv)
      pltpu.sync_copy(pid_h.at[pl.ds(off, ECH)], pidv)
      d1 = pltpu.async_copy(semb_h.at[sidv], ra, sem)
      d2 = pltpu.async_copy(cemb_h.at[cidv], rb, sem)
      d3 = pltpu.async_copy(pemb_h.at[pidv], rc, sem)
      d1.wait()
      d2.wait()
      d3.wait()

      @pl.loop(0, ECH)
      def _(r):
        for j in range(HID // 16):
          sl = pl.ds(j * 16, 16)
          ra[r, sl] = ra[r, sl] + rb[r, sl] + rc[r, sl]

      pltpu.sync_copy(ra, x_h.at[pl.ds(off, ECH)])

  return k(shape_id, colour_id, pos_id, shape_emb, col_emb, pos_emb)


def _sc_agg(x, src, dst):
  @functools.partial(
      pl.kernel,
      out_type=jax.ShapeDtypeStruct((NCORES, N, HID), jnp.float32),
      mesh=_vsc_mesh(),
      scratch_types=[
          pltpu.VMEM_SHARED((N, HID), jnp.float32),
          pltpu.VMEM((CH,), jnp.int32),
          pltpu.VMEM((CH,), jnp.int32),
          pltpu.VMEM((CH,), jnp.int32),
          pltpu.VMEM((CH, HID), jnp.float32),
          pltpu.SemaphoreType.DMA,
      ],
  )
  def k(x_h, src_h, dst_h, agg_h, aggs, srcv, dstv, rowidx, rows, sem):
    cidx = lax.axis_index("c")
    sidx = lax.axis_index("s")
    wid = sidx * NCORES + cidx

    z16 = jnp.zeros((16,), jnp.float32)

    @pl.loop(0, CH)
    def _(r):
      for j in range(HID // 16):
        rows[r, pl.ds(j * 16, 16)] = z16

    row0 = jnp.minimum(sidx * TSPAN, N - TSPAN)
    iota16 = lax.broadcasted_iota(jnp.int32, (16,), 0)

    def _fill_rowidx(base):
      for kk in range(CH // 16):
        rowidx[pl.ds(kk * 16, 16)] = iota16 + base + kk * 16

    for j in range(TSPAN // ZR):
      _fill_rowidx(row0 + j * ZR)
      pltpu.sync_copy(rows, aggs.at[rowidx])

    plsc.subcore_barrier()

    ebase = wid * EPW

    @pl.loop(0, NCH)
    def _(c):
      off = ebase + c * CH
      pltpu.sync_copy(src_h.at[pl.ds(off, CH)], srcv)
      pltpu.sync_copy(dst_h.at[pl.ds(off, CH)], dstv)
      pltpu.async_copy(x_h.at[srcv], rows, sem).wait()
      pltpu.sync_copy(rows, aggs.at[dstv], add=True)

    plsc.subcore_barrier()

    for j in range(TSPAN // ZR):
      r0 = row0 + j * ZR
      _fill_rowidx(r0)
      pltpu.async_copy(aggs.at[rowidx], rows, sem).wait()
      pltpu.sync_copy(rows, agg_h.at[cidx, pl.ds(r0, ZR)])

  return k(x, src, dst)


def _sc_counts(dst):
  @functools.partial(
      pl.kernel,
      out_type=jax.ShapeDtypeStruct((NCORES, N, HID), jnp.float32),
      mesh=_vsc_mesh(),
      scratch_types=[
          pltpu.VMEM_SHARED((N, HID), jnp.float32),
          pltpu.VMEM((CH,), jnp.int32),
          pltpu.VMEM((CH,), jnp.int32),
          pltpu.VMEM((CH, HID), jnp.float32),
          pltpu.SemaphoreType.DMA,
      ],
  )
  def k(dst_h, cnt_h, cnts, dstv, rowidx, rows, sem):
    cidx = lax.axis_index("c")
    sidx = lax.axis_index("s")
    wid = sidx * NCORES + cidx

    z16 = jnp.zeros((16,), jnp.float32)

    @pl.loop(0, CH)
    def _(r):
      for j in range(HID // 16):
        rows[r, pl.ds(j * 16, 16)] = z16

    row0 = jnp.minimum(sidx * TSPAN, N - TSPAN)
    iota16 = lax.broadcasted_iota(jnp.int32, (16,), 0)

    def _fill_rowidx(base):
      for kk in range(CH // 16):
        rowidx[pl.ds(kk * 16, 16)] = iota16 + base + kk * 16

    for j in range(TSPAN // ZR):
      _fill_rowidx(row0 + j * ZR)
      pltpu.sync_copy(rows, cnts.at[rowidx])

    one16 = jnp.full((16,), 1.0, jnp.float32)

    @pl.loop(0, CH)
    def _(r):
      for j in range(HID // 16):
        rows[r, pl.ds(j * 16, 16)] = one16

    plsc.subcore_barrier()

    ebase = wid * EPW

    @pl.loop(0, NCH)
    def _(c):
      pltpu.sync_copy(dst_h.at[pl.ds(ebase + c * CH, CH)], dstv)
      pltpu.sync_copy(rows, cnts.at[dstv], add=True)

    plsc.subcore_barrier()

    for j in range(TSPAN // ZR):
      r0 = row0 + j * ZR
      _fill_rowidx(r0)
      pltpu.async_copy(cnts.at[rowidx], rows, sem).wait()
      pltpu.sync_copy(rows, cnt_h.at[cidx, pl.ds(r0, ZR)])

  return k(dst)


def _tc_layer(x, agg, cnt, Wl, Wr, b):
  def body(x_ref, agg_ref, cnt_ref, wl_ref, wr_ref, b_ref, h_ref, st_ref):
    i = pl.program_id(0)
    c = cnt_ref[0, :, 0:1] + cnt_ref[1, :, 0:1]
    denom = jnp.maximum(c, 1.0)
    a = agg_ref[0] + agg_ref[1]
    mean = a / denom
    h = (jnp.dot(mean, wl_ref[...], preferred_element_type=jnp.float32)
         + jnp.dot(x_ref[...], wr_ref[...], preferred_element_type=jnp.float32)
         + b_ref[...])
    h_ref[...] = h

    @pl.when(i == 0)
    def _():
      st_ref[...] = jnp.zeros((8, HID), jnp.float32)

    s1 = jnp.sum(h, axis=0, keepdims=True)
    s2 = jnp.sum(h * h, axis=0, keepdims=True)
    upd = jnp.concatenate([s1, s2, jnp.zeros((6, HID), jnp.float32)], axis=0)
    st_ref[...] += upd

  return pl.pallas_call(
      body,
      grid=(GRID,),
      in_specs=[
          pl.BlockSpec((BN_BLK, HID), lambda i: (i, 0)),
          pl.BlockSpec((NCORES, BN_BLK, HID), lambda i: (0, i, 0)),
          pl.BlockSpec((NCORES, BN_BLK, HID), lambda i: (0, i, 0)),
          pl.BlockSpec((HID, HID), lambda i: (0, 0)),
          pl.BlockSpec((HID, HID), lambda i: (0, 0)),
          pl.BlockSpec((1, HID), lambda i: (0, 0)),
      ],
      out_specs=[
          pl.BlockSpec((BN_BLK, HID), lambda i: (i, 0)),
          pl.BlockSpec((8, HID), lambda i: (0, 0)),
      ],
      out_shape=[
          jax.ShapeDtypeStruct((N, HID), jnp.float32),
          jax.ShapeDtypeStruct((8, HID), jnp.float32),
      ],
  )(x, agg, cnt, Wl, Wr, b.reshape(1, HID))


def _tc_norm_relu(h, stats, g, bt):
  def body(h_ref, st_ref, g_ref, bt_ref, o_ref):
    mu = st_ref[0:1, :] * (1.0 / N)
    var = st_ref[1:2, :] * (1.0 / N) - mu * mu
    scale = g_ref[...] / jnp.sqrt(var + EPS)
    shift = bt_ref[...] - mu * scale
    o_ref[...] = jnp.maximum(h_ref[...] * scale + shift, 0.0)

  return pl.pallas_call(
      body,
      grid=(GRID,),
      in_specs=[
          pl.BlockSpec((BN_BLK, HID), lambda i: (i, 0)),
          pl.BlockSpec((8, HID), lambda i: (0, 0)),
          pl.BlockSpec((1, HID), lambda i: (0, 0)),
          pl.BlockSpec((1, HID), lambda i: (0, 0)),
      ],
      out_specs=pl.BlockSpec((BN_BLK, HID), lambda i: (i, 0)),
      out_shape=jax.ShapeDtypeStruct((N, HID), jnp.float32),
  )(h, stats, g.reshape(1, HID), bt.reshape(1, HID))


def _tc_final(h, stats, g, bt, batch2d, lin_W, lin_b):
  def body(h_ref, st_ref, g_ref, bt_ref, b_ref, lw_ref, lb_ref, o_ref,
           pool_ref):
    i = pl.program_id(0)
    mu = st_ref[0:1, :] * (1.0 / N)
    var = st_ref[1:2, :] * (1.0 / N) - mu * mu
    scale = g_ref[...] / jnp.sqrt(var + EPS)
    shift = bt_ref[...] - mu * scale
    x2 = jnp.maximum(h_ref[...] * scale + shift, 0.0)
    seg = lax.broadcasted_iota(jnp.int32, (BN_BLK, NG), 1)
    onehot = (b_ref[...] == seg).astype(jnp.float32)
    part = lax.dot_general(onehot, x2, (((0,), (0,)), ((), ())),
                           preferred_element_type=jnp.float32)

    @pl.when(i == 0)
    def _():
      pool_ref[...] = jnp.zeros((NG, HID), jnp.float32)

    pool_ref[...] += part
    o_ref[...] = jnp.dot(pool_ref[...], lw_ref[...],
                         preferred_element_type=jnp.float32) + lb_ref[...]

  return pl.pallas_call(
      body,
      grid=(GRID,),
      in_specs=[
          pl.BlockSpec((BN_BLK, HID), lambda i: (i, 0)),
          pl.BlockSpec((8, HID), lambda i: (0, 0)),
          pl.BlockSpec((1, HID), lambda i: (0, 0)),
          pl.BlockSpec((1, HID), lambda i: (0, 0)),
          pl.BlockSpec((BN_BLK, 1), lambda i: (i, 0)),
          pl.BlockSpec((HID, NCLS), lambda i: (0, 0)),
          pl.BlockSpec((1, NCLS), lambda i: (0, 0)),
      ],
      out_specs=pl.BlockSpec((NG, NCLS), lambda i: (0, 0)),
      out_shape=jax.ShapeDtypeStruct((NG, NCLS), jnp.float32),
      scratch_shapes=[pltpu.VMEM((NG, HID), jnp.float32)],
  )(h, stats, g.reshape(1, HID), bt.reshape(1, HID), batch2d, lin_W,
    lin_b.reshape(1, NCLS))


def kernel(shape_id, colour_id, pos_id, edge_index, batch,
           shape_emb, col_emb, pos_emb,
           W1l, b1l, W1r, g1, bt1,
           W2l, b2l, W2r, g2, bt2,
           lin_W, lin_b):
  src = edge_index[0].astype(jnp.int32)
  dst = edge_index[1].astype(jnp.int32)
  sid = shape_id.astype(jnp.int32)
  cid = colour_id.astype(jnp.int32)
  pid = pos_id.astype(jnp.int32)

  x = _sc_embed(sid, cid, pid, shape_emb, col_emb, pos_emb)
  cnt = _sc_counts(dst)
  agg1 = _sc_agg(x, src, dst)
  h1, st1 = _tc_layer(x, agg1, cnt, W1l, W1r, b1l)
  x2 = _tc_norm_relu(h1, st1, g1, bt1)
  agg2 = _sc_agg(x2, src, dst)
  h2, st2 = _tc_layer(x2, agg2, cnt, W2l, W2r, b2l)
  return _tc_final(h2, st2, g2, bt2, batch.reshape(N, 1).astype(jnp.int32),
                   lin_W, lin_b)


# speedup vs baseline: 5.5075x; 1.2325x over previous
import functools

import jax
import jax.numpy as jnp
from jax import lax
from jax.experimental import pallas as pl
from jax.experimental.pallas import tpu as pltpu
from jax.experimental.pallas import tpu_sc as plsc

N = 10000
E = 320000
HID = 128
NG = 64
NCLS = 2
EPS = 1e-5

NCORES = 2
NSUB = 16
NW = NCORES * NSUB

RPW = 320
ECH = 80

EPW = E // NW
CH = 80
NCH = EPW // CH
TSPAN = 640
ZR = 80

BN_BLK = 1000
GRID = N // BN_BLK


def _vsc_mesh():
  return plsc.VectorSubcoreMesh(core_axis_name="c", subcore_axis_name="s",
                                num_cores=NCORES, num_subcores=NSUB)


def _sc_embed(shape_id, colour_id, pos_id, shape_emb, col_emb, pos_emb):
  @functools.partial(
      pl.kernel,
      out_type=jax.ShapeDtypeStruct((N, HID), jnp.float32),
      mesh=_vsc_mesh(),
      scratch_types=[
          pltpu.VMEM((ECH,), jnp.int32),
          pltpu.VMEM((ECH,), jnp.int32),
          pltpu.VMEM((ECH,), jnp.int32),
          pltpu.VMEM((ECH, HID), jnp.float32),
          pltpu.VMEM((ECH, HID), jnp.float32),
          pltpu.VMEM((ECH, HID), jnp.float32),
          pltpu.SemaphoreType.DMA,
      ],
  )
  def k(sid_h, cid_h, pid_h, semb_h, cemb_h, pemb_h, x_h,
        sidv, cidv, pidv, ra, rb, rc, sem):
    wid = lax.axis_index("s") * NCORES + lax.axis_index("c")
    base = jnp.minimum(wid * RPW, N - RPW)
    for cc in range(RPW // ECH):
      off = base + cc * ECH
      pltpu.sync_copy(sid_h.at[pl.ds(off, ECH)], sidv)
      pltpu.sync_copy(cid_h.at[pl.ds(off, ECH)], cidv)
      pltpu.sync_copy(pid_h.at[pl.ds(off, ECH)], pidv)
      d1 = pltpu.async_copy(semb_h.at[sidv], ra, sem)
      d2 = pltpu.async_copy(cemb_h.at[cidv], rb, sem)
      d3 = pltpu.async_copy(pemb_h.at[pidv], rc, sem)
      d1.wait()
      d2.wait()
      d3.wait()

      @pl.loop(0, ECH)
      def _(r):
        for j in range(HID // 16):
          sl = pl.ds(j * 16, 16)
          ra[r, sl] = ra[r, sl] + rb[r, sl] + rc[r, sl]

      pltpu.sync_copy(ra, x_h.at[pl.ds(off, ECH)])

  return k(shape_id, colour_id, pos_id, shape_emb, col_emb, pos_emb)


def _sc_agg(x, src, dst):
  @functools.partial(
      pl.kernel,
      out_type=jax.ShapeDtypeStruct((NCORES, N, HID), jnp.float32),
      mesh=_vsc_mesh(),
      scratch_types=[
          pltpu.VMEM_SHARED((N, HID), jnp.float32),
          pltpu.VMEM((CH,), jnp.int32),
          pltpu.VMEM((CH,), jnp.int32),
          pltpu.VMEM((CH,), jnp.int32),
          pltpu.VMEM((CH,), jnp.int32),
          pltpu.VMEM((CH,), jnp.int32),
          pltpu.VMEM((CH, HID), jnp.float32),
          pltpu.VMEM((CH, HID), jnp.float32),
          pltpu.SemaphoreType.DMA,
          pltpu.SemaphoreType.DMA,
      ],
  )
  def k(x_h, src_h, dst_h, agg_h, aggs, srcva, dstva, srcvb, dstvb, rowidx,
        rowsa, rowsb, sema, semb):
    cidx = lax.axis_index("c")
    sidx = lax.axis_index("s")
    wid = sidx * NCORES + cidx

    z16 = jnp.zeros((16,), jnp.float32)

    @pl.loop(0, CH)
    def _(r):
      for j in range(HID // 16):
        rowsa[r, pl.ds(j * 16, 16)] = z16

    row0 = jnp.minimum(sidx * TSPAN, N - TSPAN)
    iota16 = lax.broadcasted_iota(jnp.int32, (16,), 0)

    def _fill_rowidx(base):
      for kk in range(CH // 16):
        rowidx[pl.ds(kk * 16, 16)] = iota16 + base + kk * 16

    for j in range(TSPAN // ZR):
      _fill_rowidx(row0 + j * ZR)
      pltpu.sync_copy(rowsa, aggs.at[rowidx])

    plsc.subcore_barrier()

    ebase = wid * EPW

    @pl.loop(0, NCH // 2)
    def _(t):
      offa = ebase + (2 * t) * CH
      offb = offa + CH
      pltpu.sync_copy(src_h.at[pl.ds(offa, CH)], srcva)
      pltpu.sync_copy(dst_h.at[pl.ds(offa, CH)], dstva)
      da = pltpu.async_copy(x_h.at[srcva], rowsa, sema)
      pltpu.sync_copy(src_h.at[pl.ds(offb, CH)], srcvb)
      pltpu.sync_copy(dst_h.at[pl.ds(offb, CH)], dstvb)
      db = pltpu.async_copy(x_h.at[srcvb], rowsb, semb)
      da.wait()
      pltpu.sync_copy(rowsa, aggs.at[dstva], add=True)
      db.wait()
      pltpu.sync_copy(rowsb, aggs.at[dstvb], add=True)

    if NCH % 2:
      off = ebase + (NCH - 1) * CH
      pltpu.sync_copy(src_h.at[pl.ds(off, CH)], srcva)
      pltpu.sync_copy(dst_h.at[pl.ds(off, CH)], dstva)
      pltpu.async_copy(x_h.at[srcva], rowsa, sema).wait()
      pltpu.sync_copy(rowsa, aggs.at[dstva], add=True)

    plsc.subcore_barrier()

    for j in range(TSPAN // ZR):
      r0 = row0 + j * ZR
      _fill_rowidx(r0)
      pltpu.async_copy(aggs.at[rowidx], rowsa, sema).wait()
      pltpu.sync_copy(rowsa, agg_h.at[cidx, pl.ds(r0, ZR)])

  return k(x, src, dst)


def _sc_counts(dst):
  @functools.partial(
      pl.kernel,
      out_type=jax.ShapeDtypeStruct((NCORES, N, HID), jnp.float32),
      mesh=_vsc_mesh(),
      scratch_types=[
          pltpu.VMEM_SHARED((N, HID), jnp.float32),
          pltpu.VMEM((CH,), jnp.int32),
          pltpu.VMEM((CH,), jnp.int32),
          pltpu.VMEM((CH, HID), jnp.float32),
          pltpu.SemaphoreType.DMA,
      ],
  )
  def k(dst_h, cnt_h, cnts, dstv, rowidx, rows, sem):
    cidx = lax.axis_index("c")
    sidx = lax.axis_index("s")
    wid = sidx * NCORES + cidx

    z16 = jnp.zeros((16,), jnp.float32)

    @pl.loop(0, CH)
    def _(r):
      for j in range(HID // 16):
        rows[r, pl.ds(j * 16, 16)] = z16

    row0 = jnp.minimum(sidx * TSPAN, N - TSPAN)
    iota16 = lax.broadcasted_iota(jnp.int32, (16,), 0)

    def _fill_rowidx(base):
      for kk in range(CH // 16):
        rowidx[pl.ds(kk * 16, 16)] = iota16 + base + kk * 16

    for j in range(TSPAN // ZR):
      _fill_rowidx(row0 + j * ZR)
      pltpu.sync_copy(rows, cnts.at[rowidx])

    one16 = jnp.full((16,), 1.0, jnp.float32)

    @pl.loop(0, CH)
    def _(r):
      for j in range(HID // 16):
        rows[r, pl.ds(j * 16, 16)] = one16

    plsc.subcore_barrier()

    ebase = wid * EPW

    @pl.loop(0, NCH)
    def _(c):
      pltpu.sync_copy(dst_h.at[pl.ds(ebase + c * CH, CH)], dstv)
      pltpu.sync_copy(rows, cnts.at[dstv], add=True)

    plsc.subcore_barrier()

    for j in range(TSPAN // ZR):
      r0 = row0 + j * ZR
      _fill_rowidx(r0)
      pltpu.async_copy(cnts.at[rowidx], rows, sem).wait()
      pltpu.sync_copy(rows, cnt_h.at[cidx, pl.ds(r0, ZR)])

  return k(dst)


def _tc_layer(x, agg, cnt, Wl, Wr, b):
  def body(x_ref, agg_ref, cnt_ref, wl_ref, wr_ref, b_ref, h_ref, st_ref):
    i = pl.program_id(0)
    c = cnt_ref[0, :, 0:1] + cnt_ref[1, :, 0:1]
    denom = jnp.maximum(c, 1.0)
    a = agg_ref[0] + agg_ref[1]
    mean = a / denom
    h = (jnp.dot(mean, wl_ref[...], preferred_element_type=jnp.float32)
         + jnp.dot(x_ref[...], wr_ref[...], preferred_element_type=jnp.float32)
         + b_ref[...])
    h_ref[...] = h

    @pl.when(i == 0)
    def _():
      st_ref[...] = jnp.zeros((8, HID), jnp.float32)

    s1 = jnp.sum(h, axis=0, keepdims=True)
    s2 = jnp.sum(h * h, axis=0, keepdims=True)
    upd = jnp.concatenate([s1, s2, jnp.zeros((6, HID), jnp.float32)], axis=0)
    st_ref[...] += upd

  return pl.pallas_call(
      body,
      grid=(GRID,),
      in_specs=[
          pl.BlockSpec((BN_BLK, HID), lambda i: (i, 0)),
          pl.BlockSpec((NCORES, BN_BLK, HID), lambda i: (0, i, 0)),
          pl.BlockSpec((NCORES, BN_BLK, HID), lambda i: (0, i, 0)),
          pl.BlockSpec((HID, HID), lambda i: (0, 0)),
          pl.BlockSpec((HID, HID), lambda i: (0, 0)),
          pl.BlockSpec((1, HID), lambda i: (0, 0)),
      ],
      out_specs=[
          pl.BlockSpec((BN_BLK, HID), lambda i: (i, 0)),
          pl.BlockSpec((8, HID), lambda i: (0, 0)),
      ],
      out_shape=[
          jax.ShapeDtypeStruct((N, HID), jnp.float32),
          jax.ShapeDtypeStruct((8, HID), jnp.float32),
      ],
  )(x, agg, cnt, Wl, Wr, b.reshape(1, HID))


def _tc_norm_relu(h, stats, g, bt):
  def body(h_ref, st_ref, g_ref, bt_ref, o_ref):
    mu = st_ref[0:1, :] * (1.0 / N)
    var = st_ref[1:2, :] * (1.0 / N) - mu * mu
    scale = g_ref[...] / jnp.sqrt(var + EPS)
    shift = bt_ref[...] - mu * scale
    o_ref[...] = jnp.maximum(h_ref[...] * scale + shift, 0.0)

  return pl.pallas_call(
      body,
      grid=(GRID,),
      in_specs=[
          pl.BlockSpec((BN_BLK, HID), lambda i: (i, 0)),
          pl.BlockSpec((8, HID), lambda i: (0, 0)),
          pl.BlockSpec((1, HID), lambda i: (0, 0)),
          pl.BlockSpec((1, HID), lambda i: (0, 0)),
      ],
      out_specs=pl.BlockSpec((BN_BLK, HID), lambda i: (i, 0)),
      out_shape=jax.ShapeDtypeStruct((N, HID), jnp.float32),
  )(h, stats, g.reshape(1, HID), bt.reshape(1, HID))


def _tc_final(h, stats, g, bt, batch2d, lin_W, lin_b):
  def body(h_ref, st_ref, g_ref, bt_ref, b_ref, lw_ref, lb_ref, o_ref,
           pool_ref):
    i = pl.program_id(0)
    mu = st_ref[0:1, :] * (1.0 / N)
    var = st_ref[1:2, :] * (1.0 / N) - mu * mu
    scale = g_ref[...] / jnp.sqrt(var + EPS)
    shift = bt_ref[...] - mu * scale
    x2 = jnp.maximum(h_ref[...] * scale + shift, 0.0)
    seg = lax.broadcasted_iota(jnp.int32, (BN_BLK, NG), 1)
    onehot = (b_ref[...] == seg).astype(jnp.float32)
    part = lax.dot_general(onehot, x2, (((0,), (0,)), ((), ())),
                           preferred_element_type=jnp.float32)

    @pl.when(i == 0)
    def _():
      pool_ref[...] = jnp.zeros((NG, HID), jnp.float32)

    pool_ref[...] += part
    o_ref[...] = jnp.dot(pool_ref[...], lw_ref[...],
                         preferred_element_type=jnp.float32) + lb_ref[...]

  return pl.pallas_call(
      body,
      grid=(GRID,),
      in_specs=[
          pl.BlockSpec((BN_BLK, HID), lambda i: (i, 0)),
          pl.BlockSpec((8, HID), lambda i: (0, 0)),
          pl.BlockSpec((1, HID), lambda i: (0, 0)),
          pl.BlockSpec((1, HID), lambda i: (0, 0)),
          pl.BlockSpec((BN_BLK, 1), lambda i: (i, 0)),
          pl.BlockSpec((HID, NCLS), lambda i: (0, 0)),
          pl.BlockSpec((1, NCLS), lambda i: (0, 0)),
      ],
      out_specs=pl.BlockSpec((NG, NCLS), lambda i: (0, 0)),
      out_shape=jax.ShapeDtypeStruct((NG, NCLS), jnp.float32),
      scratch_shapes=[pltpu.VMEM((NG, HID), jnp.float32)],
  )(h, stats, g.reshape(1, HID), bt.reshape(1, HID), batch2d, lin_W,
    lin_b.reshape(1, NCLS))


def kernel(shape_id, colour_id, pos_id, edge_index, batch,
           shape_emb, col_emb, pos_emb,
           W1l, b1l, W1r, g1, bt1,
           W2l, b2l, W2r, g2, bt2,
           lin_W, lin_b):
  src = edge_index[0].astype(jnp.int32)
  dst = edge_index[1].astype(jnp.int32)
  sid = shape_id.astype(jnp.int32)
  cid = colour_id.astype(jnp.int32)
  pid = pos_id.astype(jnp.int32)

  x = _sc_embed(sid, cid, pid, shape_emb, col_emb, pos_emb)
  cnt = _sc_counts(dst)
  agg1 = _sc_agg(x, src, dst)
  h1, st1 = _tc_layer(x, agg1, cnt, W1l, W1r, b1l)
  x2 = _tc_norm_relu(h1, st1, g1, bt1)
  agg2 = _sc_agg(x2, src, dst)
  h2, st2 = _tc_layer(x2, agg2, cnt, W2l, W2r, b2l)
  return _tc_final(h2, st2, g2, bt2, batch.reshape(N, 1).astype(jnp.int32),
                   lin_W, lin_b)


# speedup vs baseline: 5.6742x; 1.0303x over previous
import functools

import jax
import jax.numpy as jnp
from jax import lax
from jax.experimental import pallas as pl
from jax.experimental.pallas import tpu as pltpu
from jax.experimental.pallas import tpu_sc as plsc

N = 10000
E = 320000
HID = 128
NG = 64
NCLS = 2
EPS = 1e-5

NCORES = 2
NSUB = 16
NW = NCORES * NSUB

RPW = 320
ECH = 80

EPW = E // NW
CH = 80
NCH = EPW // CH
TSPAN = 640
ZR = 80

BN_BLK = 1000
GRID = N // BN_BLK


def _vsc_mesh():
  return plsc.VectorSubcoreMesh(core_axis_name="c", subcore_axis_name="s",
                                num_cores=NCORES, num_subcores=NSUB)


def _sc_embed(shape_id, colour_id, pos_id, shape_emb, col_emb, pos_emb):
  @functools.partial(
      pl.kernel,
      out_type=jax.ShapeDtypeStruct((N, HID), jnp.float32),
      mesh=_vsc_mesh(),
      scratch_types=[
          pltpu.VMEM((ECH,), jnp.int32),
          pltpu.VMEM((ECH,), jnp.int32),
          pltpu.VMEM((ECH,), jnp.int32),
          pltpu.VMEM((ECH,), jnp.int32),
          pltpu.VMEM((ECH,), jnp.int32),
          pltpu.VMEM((ECH,), jnp.int32),
          pltpu.VMEM((ECH, HID), jnp.float32),
          pltpu.VMEM((ECH, HID), jnp.float32),
          pltpu.VMEM((ECH, HID), jnp.float32),
          pltpu.VMEM((ECH, HID), jnp.float32),
          pltpu.VMEM((ECH, HID), jnp.float32),
          pltpu.VMEM((ECH, HID), jnp.float32),
          pltpu.SemaphoreType.DMA,
          pltpu.SemaphoreType.DMA,
      ],
  )
  def k(sid_h, cid_h, pid_h, semb_h, cemb_h, pemb_h, x_h,
        sa, ca, pa, sb, cb, pb, raa, rba, rca, rab, rbb, rcb, sma, smb):
    wid = lax.axis_index("s") * NCORES + lax.axis_index("c")
    base = jnp.minimum(wid * RPW, N - RPW)
    sets = [(sa, ca, pa, raa, rba, rca, sma),
            (sb, cb, pb, rab, rbb, rcb, smb)]

    def load_and_gather(st, off):
      sv, cv, pv, ra, rb, rc, sm = st
      pltpu.sync_copy(sid_h.at[pl.ds(off, ECH)], sv)
      pltpu.sync_copy(cid_h.at[pl.ds(off, ECH)], cv)
      pltpu.sync_copy(pid_h.at[pl.ds(off, ECH)], pv)
      return (pltpu.async_copy(semb_h.at[sv], ra, sm),
              pltpu.async_copy(cemb_h.at[cv], rb, sm),
              pltpu.async_copy(pemb_h.at[pv], rc, sm))

    nch = RPW // ECH
    descs = load_and_gather(sets[0], base)
    for cc in range(nch):
      _, _, _, ra, rb, rc, _ = sets[cc % 2]
      if cc + 1 < nch:
        ndescs = load_and_gather(sets[(cc + 1) % 2], base + (cc + 1) * ECH)
      for d in descs:
        d.wait()

      @pl.loop(0, ECH)
      def _(r):
        for j in range(HID // 16):
          sl = pl.ds(j * 16, 16)
          ra[r, sl] = ra[r, sl] + rb[r, sl] + rc[r, sl]

      pltpu.sync_copy(ra, x_h.at[pl.ds(base + cc * ECH, ECH)])
      if cc + 1 < nch:
        descs = ndescs

  return k(shape_id, colour_id, pos_id, shape_emb, col_emb, pos_emb)


def _sc_agg(x, src, dst):
  @functools.partial(
      pl.kernel,
      out_type=jax.ShapeDtypeStruct((NCORES, N, HID), jnp.float32),
      mesh=_vsc_mesh(),
      scratch_types=[
          pltpu.VMEM_SHARED((N, HID), jnp.float32),
          pltpu.VMEM((CH,), jnp.int32),
          pltpu.VMEM((CH,), jnp.int32),
          pltpu.VMEM((CH,), jnp.int32),
          pltpu.VMEM((CH,), jnp.int32),
          pltpu.VMEM((CH,), jnp.int32),
          pltpu.VMEM((CH,), jnp.int32),
          pltpu.VMEM((CH, HID), jnp.float32),
          pltpu.VMEM((CH, HID), jnp.float32),
          pltpu.SemaphoreType.DMA,
          pltpu.SemaphoreType.DMA,
      ],
  )
  def k(x_h, src_h, dst_h, agg_h, aggs, srcva, dstva, srcvb, dstvb, rowidx,
        rowidx2, rowsa, rowsb, sema, semb):
    cidx = lax.axis_index("c")
    sidx = lax.axis_index("s")
    wid = sidx * NCORES + cidx

    z16 = jnp.zeros((16,), jnp.float32)

    @pl.loop(0, CH)
    def _(r):
      for j in range(HID // 16):
        rowsa[r, pl.ds(j * 16, 16)] = z16

    row0 = jnp.minimum(sidx * TSPAN, N - TSPAN)
    iota16 = lax.broadcasted_iota(jnp.int32, (16,), 0)

    def _fill_rowidx(base):
      for kk in range(CH // 16):
        rowidx[pl.ds(kk * 16, 16)] = iota16 + base + kk * 16

    for j in range(TSPAN // ZR):
      _fill_rowidx(row0 + j * ZR)
      pltpu.sync_copy(rowsa, aggs.at[rowidx])

    plsc.subcore_barrier()

    ebase = wid * EPW

    @pl.loop(0, NCH // 2)
    def _(t):
      offa = ebase + (2 * t) * CH
      offb = offa + CH
      pltpu.sync_copy(src_h.at[pl.ds(offa, CH)], srcva)
      pltpu.sync_copy(dst_h.at[pl.ds(offa, CH)], dstva)
      da = pltpu.async_copy(x_h.at[srcva], rowsa, sema)
      pltpu.sync_copy(src_h.at[pl.ds(offb, CH)], srcvb)
      pltpu.sync_copy(dst_h.at[pl.ds(offb, CH)], dstvb)
      db = pltpu.async_copy(x_h.at[srcvb], rowsb, semb)
      da.wait()
      pltpu.sync_copy(rowsa, aggs.at[dstva], add=True)
      db.wait()
      pltpu.sync_copy(rowsb, aggs.at[dstvb], add=True)

    if NCH % 2:
      off = ebase + (NCH - 1) * CH
      pltpu.sync_copy(src_h.at[pl.ds(off, CH)], srcva)
      pltpu.sync_copy(dst_h.at[pl.ds(off, CH)], dstva)
      pltpu.async_copy(x_h.at[srcva], rowsa, sema).wait()
      pltpu.sync_copy(rowsa, aggs.at[dstva], add=True)

    plsc.subcore_barrier()

    osets = [(rowidx, rowsa, sema), (rowidx2, rowsb, semb)]

    def start_out(st, r0):
      ridx, rows, sm = st
      for kk in range(CH // 16):
        ridx[pl.ds(kk * 16, 16)] = iota16 + r0 + kk * 16
      return pltpu.async_copy(aggs.at[ridx], rows, sm)

    nz = TSPAN // ZR
    d = start_out(osets[0], row0)
    for j in range(nz):
      _, rows, _ = osets[j % 2]
      if j + 1 < nz:
        nd = start_out(osets[(j + 1) % 2], row0 + (j + 1) * ZR)
      d.wait()
      pltpu.sync_copy(rows, agg_h.at[cidx, pl.ds(row0 + j * ZR, ZR)])
      if j + 1 < nz:
        d = nd

  return k(x, src, dst)


CNTW = HID


def _sc_counts(dst):
  @functools.partial(
      pl.kernel,
      out_type=jax.ShapeDtypeStruct((NCORES, N, CNTW), jnp.float32),
      mesh=_vsc_mesh(),
      scratch_types=[
          pltpu.VMEM_SHARED((N, CNTW), jnp.float32),
          pltpu.VMEM((CH,), jnp.int32),
          pltpu.VMEM((CH,), jnp.int32),
          pltpu.VMEM((CH, CNTW), jnp.float32),
          pltpu.SemaphoreType.DMA,
      ],
  )
  def k(dst_h, cnt_h, cnts, dstv, rowidx, rows, sem):
    cidx = lax.axis_index("c")
    sidx = lax.axis_index("s")
    wid = sidx * NCORES + cidx

    z16 = jnp.zeros((16,), jnp.float32)

    @pl.loop(0, CH)
    def _(r):
      for j in range(CNTW // 16):
        rows[r, pl.ds(j * 16, 16)] = z16

    row0 = jnp.minimum(sidx * TSPAN, N - TSPAN)
    iota16 = lax.broadcasted_iota(jnp.int32, (16,), 0)

    def _fill_rowidx(base):
      for kk in range(CH // 16):
        rowidx[pl.ds(kk * 16, 16)] = iota16 + base + kk * 16

    for j in range(TSPAN // ZR):
      _fill_rowidx(row0 + j * ZR)
      pltpu.sync_copy(rows, cnts.at[rowidx])

    one16 = jnp.full((16,), 1.0, jnp.float32)

    @pl.loop(0, CH)
    def _(r):
      for j in range(CNTW // 16):
        rows[r, pl.ds(j * 16, 16)] = one16

    plsc.subcore_barrier()

    ebase = wid * EPW

    @pl.loop(0, NCH)
    def _(c):
      pltpu.sync_copy(dst_h.at[pl.ds(ebase + c * CH, CH)], dstv)
      pltpu.sync_copy(rows, cnts.at[dstv], add=True)

    plsc.subcore_barrier()

    for j in range(TSPAN // ZR):
      r0 = row0 + j * ZR
      _fill_rowidx(r0)
      pltpu.async_copy(cnts.at[rowidx], rows, sem).wait()
      pltpu.sync_copy(rows, cnt_h.at[cidx, pl.ds(r0, ZR)])

  return k(dst)


def _tc_layer(x, agg, cnt, Wl, Wr, b):
  def body(x_ref, agg_ref, cnt_ref, wl_ref, wr_ref, b_ref, h_ref, st_ref):
    i = pl.program_id(0)
    c = cnt_ref[0, :, 0:1] + cnt_ref[1, :, 0:1]
    denom = jnp.maximum(c, 1.0)
    a = agg_ref[0] + agg_ref[1]
    mean = a / denom
    h = (jnp.dot(mean, wl_ref[...], preferred_element_type=jnp.float32)
         + jnp.dot(x_ref[...], wr_ref[...], preferred_element_type=jnp.float32)
         + b_ref[...])
    h_ref[...] = h

    @pl.when(i == 0)
    def _():
      st_ref[...] = jnp.zeros((8, HID), jnp.float32)

    s1 = jnp.sum(h, axis=0, keepdims=True)
    s2 = jnp.sum(h * h, axis=0, keepdims=True)
    upd = jnp.concatenate([s1, s2, jnp.zeros((6, HID), jnp.float32)], axis=0)
    st_ref[...] += upd

  return pl.pallas_call(
      body,
      grid=(GRID,),
      in_specs=[
          pl.BlockSpec((BN_BLK, HID), lambda i: (i, 0)),
          pl.BlockSpec((NCORES, BN_BLK, HID), lambda i: (0, i, 0)),
          pl.BlockSpec((NCORES, BN_BLK, CNTW), lambda i: (0, i, 0)),
          pl.BlockSpec((HID, HID), lambda i: (0, 0)),
          pl.BlockSpec((HID, HID), lambda i: (0, 0)),
          pl.BlockSpec((1, HID), lambda i: (0, 0)),
      ],
      out_specs=[
          pl.BlockSpec((BN_BLK, HID), lambda i: (i, 0)),
          pl.BlockSpec((8, HID), lambda i: (0, 0)),
      ],
      out_shape=[
          jax.ShapeDtypeStruct((N, HID), jnp.float32),
          jax.ShapeDtypeStruct((8, HID), jnp.float32),
      ],
  )(x, agg, cnt, Wl, Wr, b.reshape(1, HID))


def _tc_norm_relu(h, stats, g, bt):
  def body(h_ref, st_ref, g_ref, bt_ref, o_ref):
    mu = st_ref[0:1, :] * (1.0 / N)
    var = st_ref[1:2, :] * (1.0 / N) - mu * mu
    scale = g_ref[...] / jnp.sqrt(var + EPS)
    shift = bt_ref[...] - mu * scale
    o_ref[...] = jnp.maximum(h_ref[...] * scale + shift, 0.0)

  return pl.pallas_call(
      body,
      grid=(GRID,),
      in_specs=[
          pl.BlockSpec((BN_BLK, HID), lambda i: (i, 0)),
          pl.BlockSpec((8, HID), lambda i: (0, 0)),
          pl.BlockSpec((1, HID), lambda i: (0, 0)),
          pl.BlockSpec((1, HID), lambda i: (0, 0)),
      ],
      out_specs=pl.BlockSpec((BN_BLK, HID), lambda i: (i, 0)),
      out_shape=jax.ShapeDtypeStruct((N, HID), jnp.float32),
  )(h, stats, g.reshape(1, HID), bt.reshape(1, HID))


def _tc_final(h, stats, g, bt, batch2d, lin_W, lin_b):
  def body(h_ref, st_ref, g_ref, bt_ref, b_ref, lw_ref, lb_ref, o_ref,
           pool_ref):
    i = pl.program_id(0)
    mu = st_ref[0:1, :] * (1.0 / N)
    var = st_ref[1:2, :] * (1.0 / N) - mu * mu
    scale = g_ref[...] / jnp.sqrt(var + EPS)
    shift = bt_ref[...] - mu * scale
    x2 = jnp.maximum(h_ref[...] * scale + shift, 0.0)
    seg = lax.broadcasted_iota(jnp.int32, (BN_BLK, NG), 1)
    onehot = (b_ref[...] == seg).astype(jnp.float32)
    part = lax.dot_general(onehot, x2, (((0,), (0,)), ((), ())),
                           preferred_element_type=jnp.float32)

    @pl.when(i == 0)
    def _():
      pool_ref[...] = jnp.zeros((NG, HID), jnp.float32)

    pool_ref[...] += part
    o_ref[...] = jnp.dot(pool_ref[...], lw_ref[...],
                         preferred_element_type=jnp.float32) + lb_ref[...]

  return pl.pallas_call(
      body,
      grid=(GRID,),
      in_specs=[
          pl.BlockSpec((BN_BLK, HID), lambda i: (i, 0)),
          pl.BlockSpec((8, HID), lambda i: (0, 0)),
          pl.BlockSpec((1, HID), lambda i: (0, 0)),
          pl.BlockSpec((1, HID), lambda i: (0, 0)),
          pl.BlockSpec((BN_BLK, 1), lambda i: (i, 0)),
          pl.BlockSpec((HID, NCLS), lambda i: (0, 0)),
          pl.BlockSpec((1, NCLS), lambda i: (0, 0)),
      ],
      out_specs=pl.BlockSpec((NG, NCLS), lambda i: (0, 0)),
      out_shape=jax.ShapeDtypeStruct((NG, NCLS), jnp.float32),
      scratch_shapes=[pltpu.VMEM((NG, HID), jnp.float32)],
  )(h, stats, g.reshape(1, HID), bt.reshape(1, HID), batch2d, lin_W,
    lin_b.reshape(1, NCLS))


def kernel(shape_id, colour_id, pos_id, edge_index, batch,
           shape_emb, col_emb, pos_emb,
           W1l, b1l, W1r, g1, bt1,
           W2l, b2l, W2r, g2, bt2,
           lin_W, lin_b):
  src = edge_index[0].astype(jnp.int32)
  dst = edge_index[1].astype(jnp.int32)
  sid = shape_id.astype(jnp.int32)
  cid = colour_id.astype(jnp.int32)
  pid = pos_id.astype(jnp.int32)

  x = _sc_embed(sid, cid, pid, shape_emb, col_emb, pos_emb)
  cnt = _sc_counts(dst)
  agg1 = _sc_agg(x, src, dst)
  h1, st1 = _tc_layer(x, agg1, cnt, W1l, W1r, b1l)
  x2 = _tc_norm_relu(h1, st1, g1, bt1)
  agg2 = _sc_agg(x2, src, dst)
  h2, st2 = _tc_layer(x2, agg2, cnt, W2l, W2r, b2l)
  return _tc_final(h2, st2, g2, bt2, batch.reshape(N, 1).astype(jnp.int32),
                   lin_W, lin_b)
